# Initial kernel scaffold; baseline (speedup 1.0000x reference)
import jax
import jax.numpy as jnp
from jax.experimental import pallas as pl


def kernel(img):
    raise NotImplementedError("write your pallas kernel here")


# speedup vs baseline: 260.8994x; 260.8994x over previous
import math

import jax
import jax.numpy as jnp
from jax.experimental import pallas as pl

NBINS = 10
P = 8
H = 512
W = 512


def _hog_body(x_ref, o_ref):
    xi = jax.lax.bitcast_convert_type(x_ref[0], jnp.int32)
    xi = (xi + jnp.int32(0x7FFF) + ((xi >> 16) & jnp.int32(1))) & jnp.int32(-65536)
    x = jax.lax.bitcast_convert_type(xi, jnp.float32)

    rows = jax.lax.broadcasted_iota(jnp.int32, (H, W), 0)
    cols = jax.lax.broadcasted_iota(jnp.int32, (H, W), 1)
    zero = jnp.float32(0.0)

    xl = jnp.where(cols == 0, zero, jnp.roll(x, 1, axis=1))
    xr = jnp.where(cols == W - 1, zero, jnp.roll(x, -1, axis=1))
    d = xl - xr
    du = jnp.where(rows == 0, zero, jnp.roll(d, 1, axis=0))
    dd = jnp.where(rows == H - 1, zero, jnp.roll(d, -1, axis=0))
    gx = du + 2.0 * d + dd

    xu = jnp.where(rows == 0, zero, jnp.roll(x, 1, axis=0))
    xd = jnp.where(rows == H - 1, zero, jnp.roll(x, -1, axis=0))
    e = xu - xd
    el = jnp.where(cols == 0, zero, jnp.roll(e, 1, axis=1))
    er = jnp.where(cols == W - 1, zero, jnp.roll(e, -1, axis=1))
    gy = el + 2.0 * e + er

    mag = jnp.sqrt(gx * gx + gy * gy)
    gx0 = gx == zero
    gy0 = gy == zero
    isint = gx0 | gy0
    neg = (gx < zero) | (gx0 & (gy < zero))
    xc = jnp.where(neg, -gy, gy)
    yc = jnp.where(neg, -gx, gx)
    binf = zero
    for k in range(1, NBINS):
        cb = jnp.float32(math.cos(k * math.pi / NBINS))
        sb = jnp.float32(math.sin(k * math.pi / NBINS))
        binf = binf + (yc * cb - xc * sb >= zero).astype(jnp.float32)
    binf = jnp.where(gx0 & gy0, zero, binf)

    pw = jnp.where(isint, jnp.float32(1.0), mag)
    qw = jnp.where(isint, zero, jnp.float32(1.0) - mag)

    cmask = [(binf == jnp.float32(k)).astype(jnp.float32) for k in range(NBINS)]

    jj = jax.lax.broadcasted_iota(jnp.int32, (W, W // P), 0)
    cc = jax.lax.broadcasted_iota(jnp.int32, (W, W // P), 1)
    pc = jnp.where(jj // P == cc, jnp.float32(1.0 / (P * P)), zero)

    strips = []
    for k in range(NBINS):
        m = pw * cmask[k] + qw * cmask[(k - 1) % NBINS]
        strips.append(m.reshape(H // P, P, W).sum(axis=1))
    stacked = jnp.concatenate(strips, axis=0)
    pooled = jnp.dot(stacked, pc, preferred_element_type=jnp.float32)
    o_ref[0] = pooled.reshape(NBINS, H // P, W // P)


def kernel(img):
    n = img.shape[0]
    out = pl.pallas_call(
        _hog_body,
        grid=(n,),
        in_specs=[pl.BlockSpec((1, H, W), lambda i: (i, 0, 0))],
        out_specs=pl.BlockSpec((1, NBINS, H // P, W // P), lambda i: (i, 0, 0, 0)),
        out_shape=jax.ShapeDtypeStruct((n, NBINS, H // P, W // P), jnp.float32),
    )(img.reshape(n, H, W))
    return out.reshape(n, -1)


# speedup vs baseline: 275.2375x; 1.0550x over previous
import math

import jax
import jax.numpy as jnp
from jax.experimental import pallas as pl

NBINS = 10
P = 8
H = 512
W = 512


def _hog_body(x_ref, o_ref):
    xi = jax.lax.bitcast_convert_type(x_ref[0], jnp.int32)
    xi = (xi + jnp.int32(0x7FFF) + ((xi >> 16) & jnp.int32(1))) & jnp.int32(-65536)
    x = jax.lax.bitcast_convert_type(xi, jnp.float32)

    rows = jax.lax.broadcasted_iota(jnp.int32, (H, W), 0)
    cols = jax.lax.broadcasted_iota(jnp.int32, (H, W), 1)
    zero = jnp.float32(0.0)

    xl = jnp.where(cols == 0, zero, jnp.roll(x, 1, axis=1))
    xr = jnp.where(cols == W - 1, zero, jnp.roll(x, -1, axis=1))
    d = xl - xr
    du = jnp.where(rows == 0, zero, jnp.roll(d, 1, axis=0))
    dd = jnp.where(rows == H - 1, zero, jnp.roll(d, -1, axis=0))
    gx = du + 2.0 * d + dd

    xu = jnp.where(rows == 0, zero, jnp.roll(x, 1, axis=0))
    xd = jnp.where(rows == H - 1, zero, jnp.roll(x, -1, axis=0))
    e = xu - xd
    el = jnp.where(cols == 0, zero, jnp.roll(e, 1, axis=1))
    er = jnp.where(cols == W - 1, zero, jnp.roll(e, -1, axis=1))
    gy = el + 2.0 * e + er

    mag = jnp.sqrt(gx * gx + gy * gy)
    gx0 = gx == zero
    gy0 = gy == zero
    isint = gx0 | gy0
    neg = (gx < zero) | (gx0 & (gy < zero))
    xc = jnp.where(neg, -gy, gy)
    yc = jnp.where(neg, -gx, gx)
    xc = jnp.where(gx0 & gy0, jnp.float32(1.0), xc)
    ustep = [jnp.float32(1.0)]
    for k in range(1, NBINS):
        cb = jnp.float32(math.cos(k * math.pi / NBINS))
        sb = jnp.float32(math.sin(k * math.pi / NBINS))
        ustep.append((yc * cb - xc * sb >= zero).astype(jnp.float32))
    ustep.append(zero)

    pw = jnp.where(isint, jnp.float32(1.0), mag)
    qw = jnp.where(isint, zero, jnp.float32(1.0) - mag)

    cmask = [ustep[k] - ustep[k + 1] for k in range(NBINS)]

    jj = jax.lax.broadcasted_iota(jnp.int32, (W, W // P), 0)
    cc = jax.lax.broadcasted_iota(jnp.int32, (W, W // P), 1)
    pc = jnp.where(jj // P == cc, jnp.float32(1.0 / (P * P)), zero)

    strips = []
    for k in range(NBINS):
        m = pw * cmask[k] + qw * cmask[(k - 1) % NBINS]
        strips.append(m.reshape(H // P, P, W).sum(axis=1))
    stacked = jnp.concatenate(strips, axis=0)
    pooled = jnp.dot(stacked, pc, preferred_element_type=jnp.float32)
    o_ref[0] = pooled.reshape(NBINS, H // P, W // P)


def kernel(img):
    n = img.shape[0]
    out = pl.pallas_call(
        _hog_body,
        grid=(n,),
        in_specs=[pl.BlockSpec((1, H, W), lambda i: (i, 0, 0))],
        out_specs=pl.BlockSpec((1, NBINS, H // P, W // P), lambda i: (i, 0, 0, 0)),
        out_shape=jax.ShapeDtypeStruct((n, NBINS, H // P, W // P), jnp.float32),
    )(img.reshape(n, H, W))
    return out.reshape(n, -1)


# speedup vs baseline: 365.3528x; 1.3274x over previous
import math

import jax
import jax.numpy as jnp
from jax.experimental import pallas as pl

NBINS = 10
P = 8
H = 512
W = 512


def _hog_body(x_ref, o_ref):
    xi = jax.lax.bitcast_convert_type(x_ref[0], jnp.int32)
    xi = (xi + jnp.int32(0x7FFF) + ((xi >> 16) & jnp.int32(1))) & jnp.int32(-65536)
    x = jax.lax.bitcast_convert_type(xi, jnp.float32)

    rows = jax.lax.broadcasted_iota(jnp.int32, (H, W), 0)
    cols = jax.lax.broadcasted_iota(jnp.int32, (H, W), 1)
    zero = jnp.float32(0.0)

    xl = jnp.where(cols == 0, zero, jnp.roll(x, 1, axis=1))
    xr = jnp.where(cols == W - 1, zero, jnp.roll(x, -1, axis=1))
    d = xl - xr
    du = jnp.where(rows == 0, zero, jnp.roll(d, 1, axis=0))
    dd = jnp.where(rows == H - 1, zero, jnp.roll(d, -1, axis=0))
    gx = du + 2.0 * d + dd

    xu = jnp.where(rows == 0, zero, jnp.roll(x, 1, axis=0))
    xd = jnp.where(rows == H - 1, zero, jnp.roll(x, -1, axis=0))
    e = xu - xd
    el = jnp.where(cols == 0, zero, jnp.roll(e, 1, axis=1))
    er = jnp.where(cols == W - 1, zero, jnp.roll(e, -1, axis=1))
    gy = el + 2.0 * e + er

    ssq = gx * gx + gy * gy
    mag = ssq * jax.lax.rsqrt(jnp.where(ssq == zero, jnp.float32(1.0), ssq))
    gx0 = gx == zero
    gy0 = gy == zero
    isint = gx0 | gy0
    neg = (gx < zero) | (gx0 & (gy < zero))
    xc = jnp.where(neg, -gy, gy)
    yc = jnp.where(neg, -gx, gx)
    xc = jnp.where(gx0 & gy0, jnp.float32(1.0), xc)
    ustep = [jnp.float32(1.0)]
    for k in range(1, NBINS):
        cb = jnp.float32(math.cos(k * math.pi / NBINS))
        sb = jnp.float32(math.sin(k * math.pi / NBINS))
        ustep.append((yc * cb - xc * sb >= zero).astype(jnp.float32))
    ustep.append(zero)

    pw = jnp.where(isint, jnp.float32(1.0), mag)
    qw = jnp.where(isint, zero, jnp.float32(1.0) - mag)

    cmask = [ustep[k] - ustep[k + 1] for k in range(NBINS)]

    jj = jax.lax.broadcasted_iota(jnp.int32, (W, W // P), 0)
    cc = jax.lax.broadcasted_iota(jnp.int32, (W, W // P), 1)
    pc = jnp.where(jj // P == cc, jnp.float32(1.0 / (P * P)), zero)

    mstack = jnp.concatenate(
        [pw * cmask[k] + qw * cmask[(k - 1) % NBINS] for k in range(NBINS)],
        axis=0)
    colp = jnp.dot(mstack, pc, preferred_element_type=jnp.float32)
    pooled = colp.reshape(NBINS * H // P, P, W // P).sum(axis=1)
    o_ref[0] = pooled.reshape(NBINS, H // P, W // P)


def kernel(img):
    n = img.shape[0]
    out = pl.pallas_call(
        _hog_body,
        grid=(n,),
        in_specs=[pl.BlockSpec((1, H, W), lambda i: (i, 0, 0))],
        out_specs=pl.BlockSpec((1, NBINS, H // P, W // P), lambda i: (i, 0, 0, 0)),
        out_shape=jax.ShapeDtypeStruct((n, NBINS, H // P, W // P), jnp.float32),
    )(img.reshape(n, H, W))
    return out.reshape(n, -1)


# speedup vs baseline: 390.2707x; 1.0682x over previous
import math

import jax
import jax.numpy as jnp
from jax.experimental import pallas as pl

NBINS = 10
P = 8
H = 512
W = 512


def _hog_body(x_ref, o_ref):
    xi = jax.lax.bitcast_convert_type(x_ref[0], jnp.int32)
    xi = (xi + jnp.int32(0x7FFF) + ((xi >> 16) & jnp.int32(1))) & jnp.int32(-65536)
    x = jax.lax.bitcast_convert_type(xi, jnp.float32)

    zero = jnp.float32(0.0)
    zrow = jnp.zeros((1, W), jnp.float32)
    zcol = jnp.zeros((H, 1), jnp.float32)

    def shl(a):
        return jnp.concatenate([a[:, 1:], zcol], axis=1)

    def shr(a):
        return jnp.concatenate([zcol, a[:, :-1]], axis=1)

    def shu(a):
        return jnp.concatenate([a[1:], zrow], axis=0)

    def shd(a):
        return jnp.concatenate([zrow, a[:-1]], axis=0)

    ii = jax.lax.broadcasted_iota(jnp.int32, (W, W), 0)
    jc = jax.lax.broadcasted_iota(jnp.int32, (W, W), 1)
    dh = (jnp.where(ii == jc - 1, jnp.float32(1.0), zero)
          - jnp.where(ii == jc + 1, jnp.float32(1.0), zero))
    d = jnp.dot(x, dh, preferred_element_type=jnp.float32)
    gx = shd(d) + 2.0 * d + shu(d)
    e = shd(x) - shu(x)
    gy = shr(e) + 2.0 * e + shl(e)

    ssq = gx * gx + gy * gy
    mag = ssq * jax.lax.rsqrt(ssq)
    gx0 = gx == zero
    gy0 = gy == zero
    isint = gx0 | gy0
    neg = (gx < zero) | (gx0 & (gy < zero))
    xc = jnp.where(neg, -gy, gy)
    yc = jnp.where(neg, -gx, gx)
    xc = jnp.where(gx0 & gy0, jnp.float32(1.0), xc)
    ustep = [jnp.float32(1.0)]
    for k in range(1, NBINS):
        cb = jnp.float32(math.cos(k * math.pi / NBINS))
        sb = jnp.float32(math.sin(k * math.pi / NBINS))
        ustep.append((yc * cb - xc * sb >= zero).astype(jnp.float32))
    ustep.append(zero)

    pw = jnp.where(isint, jnp.float32(1.0), mag)
    qw = jnp.where(isint, zero, jnp.float32(1.0) - mag)

    cmask = [ustep[k] - ustep[k + 1] for k in range(NBINS)]

    jj = jax.lax.broadcasted_iota(jnp.int32, (W, W // P), 0)
    cc = jax.lax.broadcasted_iota(jnp.int32, (W, W // P), 1)
    pc = jnp.where(jj // P == cc, jnp.float32(1.0 / (P * P)), zero)

    colps = []
    for k in range(NBINS):
        m = pw * cmask[k] + qw * cmask[(k - 1) % NBINS]
        colps.append(jnp.dot(m, pc, preferred_element_type=jnp.float32))
    colp = jnp.concatenate(colps, axis=0)
    pooled = colp.reshape(NBINS * H // P, P, W // P).sum(axis=1)
    o_ref[0] = pooled.reshape(NBINS, H // P, W // P)


def kernel(img):
    n = img.shape[0]
    out = pl.pallas_call(
        _hog_body,
        grid=(n,),
        in_specs=[pl.BlockSpec((1, H, W), lambda i: (i, 0, 0))],
        out_specs=pl.BlockSpec((1, NBINS, H // P, W // P), lambda i: (i, 0, 0, 0)),
        out_shape=jax.ShapeDtypeStruct((n, NBINS, H // P, W // P), jnp.float32),
    )(img.reshape(n, H, W))
    return out.reshape(n, -1)


# speedup vs baseline: 391.6397x; 1.0035x over previous
import math

import jax
import jax.numpy as jnp
from jax.experimental import pallas as pl
from jax.experimental.pallas import tpu as pltpu

NBINS = 10
P = 8
H = 512
W = 512


def _hog_body(x_ref, o_ref, dh_ref, sh_ref):
    x = x_ref[0]

    zero = jnp.float32(0.0)
    zrow = jnp.zeros((1, W), jnp.float32)
    zcol = jnp.zeros((H, 1), jnp.float32)

    def shl(a):
        return jnp.concatenate([a[:, 1:], zcol], axis=1)

    def shr(a):
        return jnp.concatenate([zcol, a[:, :-1]], axis=1)

    def shu(a):
        return jnp.concatenate([a[1:], zrow], axis=0)

    def shd(a):
        return jnp.concatenate([zrow, a[:-1]], axis=0)

    @pl.when(pl.program_id(0) == 0)
    def _build():
        ii = jax.lax.broadcasted_iota(jnp.int32, (W, W), 0)
        jc = jax.lax.broadcasted_iota(jnp.int32, (W, W), 1)
        dh_ref[...] = (jnp.where(ii == jc - 1, jnp.float32(1.0), zero)
                       - jnp.where(ii == jc + 1, jnp.float32(1.0), zero))
        sh_ref[...] = (jnp.where(ii == jc - 1, jnp.float32(1.0), zero)
                       + jnp.where(ii == jc, jnp.float32(2.0), zero)
                       + jnp.where(ii == jc + 1, jnp.float32(1.0), zero))

    d = jnp.dot(x, dh_ref[...], preferred_element_type=jnp.float32)
    sm = jnp.dot(x, sh_ref[...], preferred_element_type=jnp.float32)
    gx = shd(d) + 2.0 * d + shu(d)
    gy = shd(sm) - shu(sm)

    ssq = gx * gx + gy * gy
    mag = ssq * jax.lax.rsqrt(ssq)
    gx0 = gx == zero
    gy0 = gy == zero
    isint = gx0 | gy0
    neg = (gx < zero) | (gx0 & (gy < zero))
    xc = jnp.where(neg, -gy, gy)
    yc = jnp.where(neg, -gx, gx)
    xc = jnp.where(gx0 & gy0, jnp.float32(1.0), xc)
    ustep = [jnp.float32(1.0)]
    for k in range(1, NBINS):
        cb = jnp.float32(math.cos(k * math.pi / NBINS))
        sb = jnp.float32(math.sin(k * math.pi / NBINS))
        ustep.append((yc * cb - xc * sb >= zero).astype(jnp.float32))
    ustep.append(zero)

    pw = jnp.where(isint, jnp.float32(1.0), mag)
    qw = jnp.where(isint, zero, jnp.float32(1.0) - mag)

    cmask = [ustep[k] - ustep[k + 1] for k in range(NBINS)]

    jj = jax.lax.broadcasted_iota(jnp.int32, (W, W // P), 0)
    cc = jax.lax.broadcasted_iota(jnp.int32, (W, W // P), 1)
    pc = jnp.where(jj // P == cc, jnp.float32(1.0 / (P * P)), zero)

    colps = []
    for k in range(NBINS):
        m = pw * cmask[k] + qw * cmask[(k - 1) % NBINS]
        colps.append(jnp.dot(m, pc, preferred_element_type=jnp.float32))
    colp = jnp.concatenate(colps, axis=0)
    pooled = colp.reshape(NBINS * H // P, P, W // P).sum(axis=1)
    o_ref[0] = pooled.reshape(NBINS, H // P, W // P)


def kernel(img):
    n = img.shape[0]
    out = pl.pallas_call(
        _hog_body,
        grid=(n,),
        in_specs=[pl.BlockSpec((1, H, W), lambda i: (i, 0, 0))],
        out_specs=pl.BlockSpec((1, NBINS, H // P, W // P), lambda i: (i, 0, 0, 0)),
        out_shape=jax.ShapeDtypeStruct((n, NBINS, H // P, W // P), jnp.float32),
        scratch_shapes=[pltpu.VMEM((W, W), jnp.float32),
                        pltpu.VMEM((W, W), jnp.float32)],
    )(img.reshape(n, H, W))
    return out.reshape(n, -1)


# speedup vs baseline: 398.9162x; 1.0186x over previous
import math

import jax
import jax.numpy as jnp
from jax.experimental import pallas as pl
from jax.experimental.pallas import tpu as pltpu

NBINS = 10
P = 8
H = 512
W = 512


def _hog_body(x_ref, dh_ref, sh_ref, pc_ref, o_ref):
    x = x_ref[0]

    zero = jnp.float32(0.0)
    zrow = jnp.zeros((1, W), jnp.float32)
    zcol = jnp.zeros((H, 1), jnp.float32)

    def shl(a):
        return jnp.concatenate([a[:, 1:], zcol], axis=1)

    def shr(a):
        return jnp.concatenate([zcol, a[:, :-1]], axis=1)

    def shu(a):
        return jnp.concatenate([a[1:], zrow], axis=0)

    def shd(a):
        return jnp.concatenate([zrow, a[:-1]], axis=0)

    d = jnp.dot(x, dh_ref[...], preferred_element_type=jnp.float32)
    sm = jnp.dot(x, sh_ref[...], preferred_element_type=jnp.float32)
    gx = shd(d) + 2.0 * d + shu(d)
    gy = shd(sm) - shu(sm)

    ssq = gx * gx + gy * gy
    mag = ssq * jax.lax.rsqrt(ssq)
    gx0 = gx == zero
    gy0 = gy == zero
    isint = gx0 | gy0
    neg = (gx < zero) | (gx0 & (gy < zero))
    xc = jnp.where(neg, -gy, gy)
    yc = jnp.where(neg, -gx, gx)
    xc = jnp.where(gx0 & gy0, jnp.float32(1.0), xc)
    ustep = [jnp.float32(1.0)]
    for k in range(1, NBINS):
        cb = jnp.float32(math.cos(k * math.pi / NBINS))
        sb = jnp.float32(math.sin(k * math.pi / NBINS))
        ustep.append((yc * cb - xc * sb >= zero).astype(jnp.float32))
    ustep.append(zero)

    pw = jnp.where(isint, jnp.float32(1.0), mag)
    qw = jnp.where(isint, zero, jnp.float32(1.0) - mag)

    cmask = [ustep[k] - ustep[k + 1] for k in range(NBINS)]

    pc = pc_ref[...]

    colps = []
    for k in range(NBINS):
        m = pw * cmask[k] + qw * cmask[(k - 1) % NBINS]
        colps.append(jnp.dot(m, pc, preferred_element_type=jnp.float32))
    colp = jnp.concatenate(colps, axis=0)
    pooled = colp.reshape(NBINS * H // P, P, W // P).sum(axis=1)
    o_ref[0] = pooled.reshape(NBINS, H // P, W // P)


import numpy as _np
import functools


@functools.lru_cache(maxsize=1)
def _dh_mat():
    return jnp.asarray(_np.eye(W, k=1, dtype=_np.float32)
                       - _np.eye(W, k=-1, dtype=_np.float32))


@functools.lru_cache(maxsize=1)
def _sh_mat():
    return jnp.asarray(_np.eye(W, k=1, dtype=_np.float32)
                       + 2.0 * _np.eye(W, dtype=_np.float32)
                       + _np.eye(W, k=-1, dtype=_np.float32))


@functools.lru_cache(maxsize=1)
def _pc_mat():
    return jnp.asarray(_np.kron(_np.eye(W // P, dtype=_np.float32),
                                _np.full((P, 1), 1.0 / (P * P), _np.float32)))


def kernel(img):
    n = img.shape[0]
    out = pl.pallas_call(
        _hog_body,
        grid=(n,),
        in_specs=[pl.BlockSpec((1, H, W), lambda i: (i, 0, 0)),
                  pl.BlockSpec((W, W), lambda i: (0, 0)),
                  pl.BlockSpec((W, W), lambda i: (0, 0)),
                  pl.BlockSpec((W, W // P), lambda i: (0, 0))],
        out_specs=pl.BlockSpec((1, NBINS, H // P, W // P), lambda i: (i, 0, 0, 0)),
        out_shape=jax.ShapeDtypeStruct((n, NBINS, H // P, W // P), jnp.float32),
    )(img.reshape(n, H, W), _dh_mat(), _sh_mat(), _pc_mat())
    return out.reshape(n, -1)


# speedup vs baseline: 404.6442x; 1.0144x over previous
import math

import jax
import jax.numpy as jnp
from jax.experimental import pallas as pl
from jax.experimental.pallas import tpu as pltpu

NBINS = 10
P = 8
H = 512
W = 512


def _hog_body(x_ref, dh_ref, sh_ref, pc_ref, o_ref):
  for _b in range(2):
      x = x_ref[_b]

      zero = jnp.float32(0.0)
      zrow = jnp.zeros((1, W), jnp.float32)
      zcol = jnp.zeros((H, 1), jnp.float32)

      def shl(a):
          return jnp.concatenate([a[:, 1:], zcol], axis=1)

      def shr(a):
          return jnp.concatenate([zcol, a[:, :-1]], axis=1)

      def shu(a):
          return jnp.concatenate([a[1:], zrow], axis=0)

      def shd(a):
          return jnp.concatenate([zrow, a[:-1]], axis=0)

      d = jnp.dot(x, dh_ref[...], preferred_element_type=jnp.float32)
      sm = jnp.dot(x, sh_ref[...], preferred_element_type=jnp.float32)
      gx = shd(d) + 2.0 * d + shu(d)
      gy = shd(sm) - shu(sm)

      ssq = gx * gx + gy * gy
      mag = ssq * jax.lax.rsqrt(ssq)
      gx0 = gx == zero
      gy0 = gy == zero
      isint = gx0 | gy0
      neg = (gx < zero) | (gx0 & (gy < zero))
      xc = jnp.where(neg, -gy, gy)
      yc = jnp.where(neg, -gx, gx)
      xc = jnp.where(gx0 & gy0, jnp.float32(1.0), xc)
      ustep = [jnp.float32(1.0)]
      for k in range(1, NBINS):
          cb = jnp.float32(math.cos(k * math.pi / NBINS))
          sb = jnp.float32(math.sin(k * math.pi / NBINS))
          ustep.append((yc * cb - xc * sb >= zero).astype(jnp.float32))
      ustep.append(zero)

      pw = jnp.where(isint, jnp.float32(1.0), mag)
      qw = jnp.where(isint, zero, jnp.float32(1.0) - mag)

      cmask = [ustep[k] - ustep[k + 1] for k in range(NBINS)]

      pc = pc_ref[...]

      colps = []
      for k in range(NBINS):
          m = pw * cmask[k] + qw * cmask[(k - 1) % NBINS]
          colps.append(jnp.dot(m, pc, preferred_element_type=jnp.float32))
      colp = jnp.concatenate(colps, axis=0)
      pooled = colp.reshape(NBINS * H // P, P, W // P).sum(axis=1)
      o_ref[_b] = pooled.reshape(NBINS, H // P, W // P)


import numpy as _np
import functools


@functools.lru_cache(maxsize=1)
def _dh_mat():
    return jnp.asarray(_np.eye(W, k=1, dtype=_np.float32)
                       - _np.eye(W, k=-1, dtype=_np.float32))


@functools.lru_cache(maxsize=1)
def _sh_mat():
    return jnp.asarray(_np.eye(W, k=1, dtype=_np.float32)
                       + 2.0 * _np.eye(W, dtype=_np.float32)
                       + _np.eye(W, k=-1, dtype=_np.float32))


@functools.lru_cache(maxsize=1)
def _pc_mat():
    return jnp.asarray(_np.kron(_np.eye(W // P, dtype=_np.float32),
                                _np.full((P, 1), 1.0 / (P * P), _np.float32)))


def kernel(img):
    n = img.shape[0]
    out = pl.pallas_call(
        _hog_body,
        grid=(n // 2,),
        in_specs=[pl.BlockSpec((2, H, W), lambda i: (i, 0, 0)),
                  pl.BlockSpec((W, W), lambda i: (0, 0)),
                  pl.BlockSpec((W, W), lambda i: (0, 0)),
                  pl.BlockSpec((W, W // P), lambda i: (0, 0))],
        out_specs=pl.BlockSpec((2, NBINS, H // P, W // P), lambda i: (i, 0, 0, 0)),
        out_shape=jax.ShapeDtypeStruct((n, NBINS, H // P, W // P), jnp.float32),
    )(img.reshape(n, H, W), _dh_mat(), _sh_mat(), _pc_mat())
    return out.reshape(n, -1)


# speedup vs baseline: 405.2098x; 1.0014x over previous
import functools
import math

import jax
import jax.numpy as jnp
import numpy as np
from jax.experimental import pallas as pl

NBINS = 10
P = 8
H = 512
W = 512
B = 2


def _hog_body(x_ref, dh_ref, sh_ref, pc_ref, o_ref):
  for b in range(B):
      x = x_ref[b]
      zero = jnp.float32(0.0)
      zrow = jnp.zeros((1, W), jnp.float32)

      def shu(a):
          return jnp.concatenate([a[1:], zrow], axis=0)

      def shd(a):
          return jnp.concatenate([zrow, a[:-1]], axis=0)

      d = jnp.dot(x, dh_ref[...], preferred_element_type=jnp.float32)
      sm = jnp.dot(x, sh_ref[...], preferred_element_type=jnp.float32)
      gx = shd(d) + 2.0 * d + shu(d)
      gy = shd(sm) - shu(sm)

      ssq = gx * gx + gy * gy
      mag = ssq * jax.lax.rsqrt(ssq)

      gx0 = gx == zero
      gy0 = gy == zero
      isint = gx0 | gy0
      neg = (gx < zero) | (gx0 & (gy < zero))
      xc = jnp.where(neg, -gy, gy)
      yc = jnp.where(neg, -gx, gx)
      xc = jnp.where(gx0 & gy0, jnp.float32(1.0), xc)
      ustep = [jnp.float32(1.0)]
      for k in range(1, NBINS):
          cb = jnp.float32(math.cos(k * math.pi / NBINS))
          sb = jnp.float32(math.sin(k * math.pi / NBINS))
          ustep.append((yc * cb - xc * sb >= zero).astype(jnp.float32))
      ustep.append(zero)
      cmask = [ustep[k] - ustep[k + 1] for k in range(NBINS)]

      pw = jnp.where(isint, jnp.float32(1.0), mag)
      qw = jnp.where(isint, zero, jnp.float32(1.0) - mag)

      pc = pc_ref[...]
      colps = []
      for k in range(NBINS):
          m = pw * cmask[k] + qw * cmask[(k - 1) % NBINS]
          colps.append(jnp.dot(m, pc, preferred_element_type=jnp.float32))
      colp = jnp.concatenate(colps, axis=0)
      pooled = colp.reshape(NBINS * H // P, P, W // P).sum(axis=1)
      o_ref[b] = pooled.reshape(NBINS, H // P, W // P)


@functools.lru_cache(maxsize=1)
def _dh_mat():
    return jnp.asarray(np.eye(W, k=1, dtype=np.float32)
                       - np.eye(W, k=-1, dtype=np.float32))


@functools.lru_cache(maxsize=1)
def _sh_mat():
    return jnp.asarray(np.eye(W, k=1, dtype=np.float32)
                       + 2.0 * np.eye(W, dtype=np.float32)
                       + np.eye(W, k=-1, dtype=np.float32))


@functools.lru_cache(maxsize=1)
def _pc_mat():
    return jnp.asarray(np.kron(np.eye(W // P, dtype=np.float32),
                               np.full((P, 1), 1.0 / (P * P), np.float32)))


def kernel(img):
    n = img.shape[0]
    out = pl.pallas_call(
        _hog_body,
        grid=(n // B,),
        in_specs=[pl.BlockSpec((B, H, W), lambda i: (i, 0, 0)),
                  pl.BlockSpec((W, W), lambda i: (0, 0)),
                  pl.BlockSpec((W, W), lambda i: (0, 0)),
                  pl.BlockSpec((W, W // P), lambda i: (0, 0))],
        out_specs=pl.BlockSpec((B, NBINS, H // P, W // P),
                               lambda i: (i, 0, 0, 0)),
        out_shape=jax.ShapeDtypeStruct((n, NBINS, H // P, W // P),
                                       jnp.float32),
    )(img.reshape(n, H, W), _dh_mat(), _sh_mat(), _pc_mat())
    return out.reshape(n, -1)
